# baseline (speedup 1.0000x reference)
import functools

import jax
import jax.numpy as jnp
from jax import lax
from jax.experimental import pallas as pl
from jax.experimental.pallas import tpu as pltpu
from jax.experimental.pallas import tpu_sc as plsc

NUM_CORES = 2
NUM_SUBCORES = 16
LANES = 16
NW = NUM_CORES * NUM_SUBCORES

B_PER_W = 512
CHUNK = 128
NCHUNK = B_PER_W // CHUNK
NG = CHUNK // LANES


def _mf_body(uidx_hbm, iidx_hbm, ut_hbm, it_hbm, out_hbm,
             idx_r, idx_s, data_u, data_i, out_v, sem):
    d_model = ut_hbm.shape[0]
    wid = lax.axis_index("s") * NUM_CORES + lax.axis_index("c")
    base = wid * B_PER_W

    pltpu.sync_copy(uidx_hbm.at[pl.ds(base, B_PER_W)], idx_r)
    pltpu.sync_copy(iidx_hbm.at[pl.ds(base, B_PER_W)], idx_s)

    def chunk_body(c, carry):
        iu = idx_r.at[pl.ds(c * CHUNK, CHUNK)]
        ii = idx_s.at[pl.ds(c * CHUNK, CHUNK)]
        for d in range(d_model):
            pltpu.async_copy(ut_hbm.at[d].at[iu],
                             data_u.at[pl.ds(d * CHUNK, CHUNK)], sem)
            pltpu.async_copy(it_hbm.at[d].at[ii],
                             data_i.at[pl.ds(d * CHUNK, CHUNK)], sem)

        drain = pltpu.make_async_copy(
            out_hbm.at[pl.ds(0, d_model * CHUNK)], data_u, sem)
        drain.wait()
        drain.wait()

        def gbody(g, carry2):
            acc = jnp.zeros((LANES,), jnp.float32)
            for d in range(d_model):
                u = data_u[pl.ds(d * CHUNK + g * LANES, LANES)]
                w = data_i[pl.ds(d * CHUNK + g * LANES, LANES)]
                acc = acc + u * w
            out_v[pl.ds(c * CHUNK + g * LANES, LANES)] = acc
            return carry2

        lax.fori_loop(0, NG, gbody, 0)
        return carry

    lax.fori_loop(0, NCHUNK, chunk_body, 0)

    pltpu.sync_copy(out_v, out_hbm.at[pl.ds(base, B_PER_W)])


@jax.jit
def kernel(user_idx, item_idx, user_table, item_table):
    batch = user_idx.shape[0]
    v_rows, d_model = user_table.shape
    assert batch == NW * B_PER_W, batch

    ut = user_table.T
    it = item_table.T
    uidx = user_idx.astype(jnp.int32)
    iidx = item_idx.astype(jnp.int32)

    mesh = plsc.VectorSubcoreMesh(core_axis_name="c", subcore_axis_name="s",
                                  num_cores=NUM_CORES,
                                  num_subcores=NUM_SUBCORES)
    kfn = pl.kernel(
        _mf_body,
        out_type=jax.ShapeDtypeStruct((batch,), jnp.float32),
        mesh=mesh,
        compiler_params=pltpu.CompilerParams(needs_layout_passes=False,
                                             use_tc_tiling_on_sc=False),
        scratch_types=[
            pltpu.VMEM((B_PER_W,), jnp.int32),
            pltpu.VMEM((B_PER_W,), jnp.int32),
            pltpu.VMEM((d_model * CHUNK,), jnp.float32),
            pltpu.VMEM((d_model * CHUNK,), jnp.float32),
            pltpu.VMEM((B_PER_W,), jnp.float32),
            pltpu.SemaphoreType.DMA,
        ],
    )
    return kfn(uidx, iidx, ut, it)


# speedup vs baseline: 9.4494x; 9.4494x over previous
import functools

import jax
import jax.numpy as jnp
from jax import lax
from jax.experimental import pallas as pl
from jax.experimental.pallas import tpu as pltpu
from jax.experimental.pallas import tpu_sc as plsc

NUM_CORES = 2
NUM_SUBCORES = 16
LANES = 16
NW = NUM_CORES * NUM_SUBCORES

B_PER_W = 512
CHUNK = 128
NCHUNK = B_PER_W // CHUNK
NG = CHUNK // LANES
DPAD = 128


def _mf_body(uidx_hbm, iidx_hbm, ut_hbm, it_hbm, out_hbm,
             idx_r, idx_s, data_u, data_i, out_v, sem):
    d_model = 64
    wid = lax.axis_index("s") * NUM_CORES + lax.axis_index("c")
    base = wid * B_PER_W

    pltpu.sync_copy(uidx_hbm.at[pl.ds(base, B_PER_W)], idx_r)
    pltpu.sync_copy(iidx_hbm.at[pl.ds(base, B_PER_W)], idx_s)

    iota16 = lax.iota(jnp.int32, LANES)

    def chunk_body(c, carry):
        pltpu.async_copy(ut_hbm.at[idx_r.at[pl.ds(c * CHUNK, CHUNK)]],
                         data_u, sem)
        pltpu.async_copy(it_hbm.at[idx_s.at[pl.ds(c * CHUNK, CHUNK)]],
                         data_i, sem)
        pltpu.make_async_copy(ut_hbm.at[pl.ds(0, CHUNK)], data_u, sem).wait()
        pltpu.make_async_copy(ut_hbm.at[pl.ds(0, CHUNK)], data_i, sem).wait()

        def gbody(g, carry2):
            row16 = g * LANES + iota16
            acc = jnp.zeros((LANES,), jnp.float32)
            for d in range(d_model):
                dv = jnp.full((LANES,), d, jnp.int32)
                u = plsc.load_gather(data_u, [row16, dv])
                w = plsc.load_gather(data_i, [row16, dv])
                acc = acc + u * w
            out_v[pl.ds(c * CHUNK + g * LANES, LANES)] = acc
            return carry2

        lax.fori_loop(0, NG, gbody, 0)
        return carry

    lax.fori_loop(0, NCHUNK, chunk_body, 0)

    pltpu.sync_copy(out_v, out_hbm.at[pl.ds(base, B_PER_W)])


@jax.jit
def kernel(user_idx, item_idx, user_table, item_table):
    batch = user_idx.shape[0]
    v_rows, d_model = user_table.shape
    assert batch == NW * B_PER_W, batch
    assert d_model == 64

    ut = jnp.pad(user_table, ((0, 0), (0, DPAD - d_model)))
    it = jnp.pad(item_table, ((0, 0), (0, DPAD - d_model)))
    uidx = user_idx.astype(jnp.int32)
    iidx = item_idx.astype(jnp.int32)

    mesh = plsc.VectorSubcoreMesh(core_axis_name="c", subcore_axis_name="s",
                                  num_cores=NUM_CORES,
                                  num_subcores=NUM_SUBCORES)
    kfn = pl.kernel(
        _mf_body,
        out_type=jax.ShapeDtypeStruct((batch,), jnp.float32),
        mesh=mesh,
        compiler_params=pltpu.CompilerParams(needs_layout_passes=False,
                                             use_tc_tiling_on_sc=False),
        scratch_types=[
            pltpu.VMEM((B_PER_W,), jnp.int32),
            pltpu.VMEM((B_PER_W,), jnp.int32),
            pltpu.VMEM((CHUNK, DPAD), jnp.float32),
            pltpu.VMEM((CHUNK, DPAD), jnp.float32),
            pltpu.VMEM((B_PER_W,), jnp.float32),
            pltpu.SemaphoreType.DMA,
        ],
    )
    return kfn(uidx, iidx, ut, it)


# speedup vs baseline: 9.4829x; 1.0036x over previous
import functools

import jax
import jax.numpy as jnp
from jax import lax
from jax.experimental import pallas as pl
from jax.experimental.pallas import tpu as pltpu
from jax.experimental.pallas import tpu_sc as plsc

NUM_CORES = 2
NUM_SUBCORES = 16
LANES = 16
NW = NUM_CORES * NUM_SUBCORES

B_PER_W = 512
CHUNK = 128
NCHUNK = B_PER_W // CHUNK
NG = CHUNK // LANES
DPAD = 128


def _mf_body(uidx_hbm, iidx_hbm, ut_hbm, it_hbm, out_hbm,
             idx_r, idx_s, data_u, data_i, out_v, sem_a, sem_b):
    d_model = 64
    wid = lax.axis_index("s") * NUM_CORES + lax.axis_index("c")
    base = wid * B_PER_W

    pltpu.sync_copy(uidx_hbm.at[pl.ds(base, B_PER_W)], idx_r)
    pltpu.sync_copy(iidx_hbm.at[pl.ds(base, B_PER_W)], idx_s)

    iota16 = lax.iota(jnp.int32, LANES)
    sems = (sem_a, sem_b)

    def fire(c):
        p = c % 2
        pltpu.async_copy(ut_hbm.at[idx_r.at[pl.ds(c * CHUNK, CHUNK)]],
                         data_u.at[p], sems[p])
        pltpu.async_copy(it_hbm.at[idx_s.at[pl.ds(c * CHUNK, CHUNK)]],
                         data_i.at[p], sems[p])

    fire(0)
    for c in range(NCHUNK):
        p = c % 2
        if c + 1 < NCHUNK:
            fire(c + 1)
        pltpu.make_async_copy(ut_hbm.at[pl.ds(0, CHUNK)],
                              data_u.at[p], sems[p]).wait()
        pltpu.make_async_copy(ut_hbm.at[pl.ds(0, CHUNK)],
                              data_i.at[p], sems[p]).wait()

        pv = jnp.full((LANES,), p, jnp.int32)

        def gbody(g, carry2, pv=pv, c=c):
            row16 = g * LANES + iota16
            acc = jnp.zeros((LANES,), jnp.float32)
            for d in range(d_model):
                dv = jnp.full((LANES,), d, jnp.int32)
                u = plsc.load_gather(data_u, [pv, row16, dv])
                w = plsc.load_gather(data_i, [pv, row16, dv])
                acc = acc + u * w
            out_v[pl.ds(c * CHUNK + g * LANES, LANES)] = acc
            return carry2

        lax.fori_loop(0, NG, gbody, 0)

    pltpu.sync_copy(out_v, out_hbm.at[pl.ds(base, B_PER_W)])


@jax.jit
def kernel(user_idx, item_idx, user_table, item_table):
    batch = user_idx.shape[0]
    v_rows, d_model = user_table.shape
    assert batch == NW * B_PER_W, batch
    assert d_model == 64

    ut = jnp.pad(user_table, ((0, 0), (0, DPAD - d_model)))
    it = jnp.pad(item_table, ((0, 0), (0, DPAD - d_model)))
    uidx = user_idx.astype(jnp.int32)
    iidx = item_idx.astype(jnp.int32)

    mesh = plsc.VectorSubcoreMesh(core_axis_name="c", subcore_axis_name="s",
                                  num_cores=NUM_CORES,
                                  num_subcores=NUM_SUBCORES)
    kfn = pl.kernel(
        _mf_body,
        out_type=jax.ShapeDtypeStruct((batch,), jnp.float32),
        mesh=mesh,
        compiler_params=pltpu.CompilerParams(needs_layout_passes=False,
                                             use_tc_tiling_on_sc=False),
        scratch_types=[
            pltpu.VMEM((B_PER_W,), jnp.int32),
            pltpu.VMEM((B_PER_W,), jnp.int32),
            pltpu.VMEM((2, CHUNK, DPAD), jnp.float32),
            pltpu.VMEM((2, CHUNK, DPAD), jnp.float32),
            pltpu.VMEM((B_PER_W,), jnp.float32),
            pltpu.SemaphoreType.DMA,
            pltpu.SemaphoreType.DMA,
        ],
    )
    return kfn(uidx, iidx, ut, it)


# speedup vs baseline: 10.7716x; 1.1359x over previous
import functools

import jax
import jax.numpy as jnp
from jax import lax
from jax.experimental import pallas as pl
from jax.experimental.pallas import tpu as pltpu
from jax.experimental.pallas import tpu_sc as plsc

NUM_CORES = 2
NUM_SUBCORES = 16
LANES = 16
NW = NUM_CORES * NUM_SUBCORES

B_PER_W = 512
CHUNK = 128
NCHUNK = B_PER_W // CHUNK
NG = CHUNK // LANES
DCAT = 128


def _mf_body(uidx_hbm, iidx_hbm, tab_hbm, out_hbm,
             idx_r, idx_s, data_u, data_i, out_v, sem_a, sem_b):
    d_model = DCAT // 2
    wid = lax.axis_index("s") * NUM_CORES + lax.axis_index("c")
    base = wid * B_PER_W

    pltpu.sync_copy(uidx_hbm.at[pl.ds(base, B_PER_W)], idx_r)
    pltpu.sync_copy(iidx_hbm.at[pl.ds(base, B_PER_W)], idx_s)

    iota16 = lax.iota(jnp.int32, LANES)
    sems = (sem_a, sem_b)

    def fire(c):
        p = c % 2
        pltpu.async_copy(tab_hbm.at[idx_r.at[pl.ds(c * CHUNK, CHUNK)]],
                         data_u.at[p], sems[p])
        pltpu.async_copy(tab_hbm.at[idx_s.at[pl.ds(c * CHUNK, CHUNK)]],
                         data_i.at[p], sems[p])

    fire(0)
    for c in range(NCHUNK):
        p = c % 2
        if c + 1 < NCHUNK:
            fire(c + 1)
        pltpu.make_async_copy(tab_hbm.at[pl.ds(0, CHUNK)],
                              data_u.at[p], sems[p]).wait()
        pltpu.make_async_copy(tab_hbm.at[pl.ds(0, CHUNK)],
                              data_i.at[p], sems[p]).wait()

        pv = jnp.full((LANES,), p, jnp.int32)

        def gbody(g, carry, pv=pv, c=c):
            row16 = g * LANES + iota16
            acc = jnp.zeros((LANES,), jnp.float32)
            for d in range(d_model):
                dv = jnp.full((LANES,), d, jnp.int32)
                u = plsc.load_gather(data_u, [pv, row16, dv])
                w = plsc.load_gather(data_i, [pv, row16, dv + d_model])
                acc = acc + u * w
            out_v[pl.ds(c * CHUNK + g * LANES, LANES)] = acc
            return carry

        lax.fori_loop(0, NG, gbody, 0)

    pltpu.sync_copy(out_v, out_hbm.at[pl.ds(base, B_PER_W)])


@jax.jit
def kernel(user_idx, item_idx, user_table, item_table):
    batch = user_idx.shape[0]
    v_rows, d_model = user_table.shape
    assert batch == NW * B_PER_W, batch
    assert 2 * d_model == DCAT

    tab = jnp.concatenate([user_table, item_table], axis=1)
    uidx = user_idx.astype(jnp.int32)
    iidx = item_idx.astype(jnp.int32)

    mesh = plsc.VectorSubcoreMesh(core_axis_name="c", subcore_axis_name="s",
                                  num_cores=NUM_CORES,
                                  num_subcores=NUM_SUBCORES)
    kfn = pl.kernel(
        _mf_body,
        out_type=jax.ShapeDtypeStruct((batch,), jnp.float32),
        mesh=mesh,
        compiler_params=pltpu.CompilerParams(needs_layout_passes=False,
                                             use_tc_tiling_on_sc=False),
        scratch_types=[
            pltpu.VMEM((B_PER_W,), jnp.int32),
            pltpu.VMEM((B_PER_W,), jnp.int32),
            pltpu.VMEM((2, CHUNK, DCAT), jnp.float32),
            pltpu.VMEM((2, CHUNK, DCAT), jnp.float32),
            pltpu.VMEM((B_PER_W,), jnp.float32),
            pltpu.SemaphoreType.DMA,
            pltpu.SemaphoreType.DMA,
        ],
    )
    return kfn(uidx, iidx, tab)
